# baseline (speedup 1.0000x reference)
import functools

import jax
import jax.numpy as jnp
from jax import lax
from jax.experimental import pallas as pl
from jax.experimental.pallas import tpu as pltpu
from jax.experimental.pallas import tpu_sc as plsc


_F32 = jnp.float32
_I32 = jnp.int32


def _prologue_body(bn_ref, ts_ref, el_ref, lagcat_ref, elcat_ref, n0_ref, n1_ref):
    t = ts_ref[...]
    rows, cols = t.shape
    lanes = lax.broadcasted_iota(_I32, (rows, cols), 1)
    tp = jnp.where(lanes >= 1, jnp.roll(t, 1, axis=1), t)
    m = jnp.where(t != tp, tp, -1.0)
    k = 1
    while k < cols:
        m = jnp.maximum(m, jnp.where(lanes >= k, jnp.roll(m, k, axis=1), -1.0))
        k *= 2
    prev = jnp.where(m < 0.0, t, m)
    lag = jnp.clip((t - prev) / 60000.0, 0.0, 1440.0)
    lagcat_ref[...] = jnp.where(
        lag < 6.0, lag.astype(_I32), ((lag - 1.0) / 10.0).astype(_I32) + 6
    )
    e = el_ref[...]
    elcat_ref[...] = jnp.clip(e.astype(_I32) + 1, 0, 300)
    e_num = jnp.clip(e, 0.0, 300.0)
    lf = jnp.log1p(lag)
    s0 = jnp.sqrt(bn_ref[1, 0] + 1e-5)
    s1 = jnp.sqrt(bn_ref[1, 1] + 1e-5)
    n0_ref[...] = (lf - bn_ref[0, 0]) / s0 * bn_ref[2, 0] + bn_ref[3, 0]
    n1_ref[...] = (e_num - bn_ref[0, 1]) / s1 * bn_ref[2, 1] + bn_ref[3, 1]


def _sc_gather(rid, lid, eid, pid, resp_t, lag_t, el_t, pos_t):
    n = rid.shape[0]
    num_cores, num_subcores = 2, 16
    nw = num_cores * num_subcores
    per_w = n // nw
    ch = 128
    n_ch = per_w // ch
    mesh = plsc.VectorSubcoreMesh(
        core_axis_name="c", subcore_axis_name="s",
        num_cores=num_cores, num_subcores=num_subcores,
    )
    widths = (resp_t.shape[1], lag_t.shape[1], el_t.shape[1], pos_t.shape[1])
    out_type = tuple(jax.ShapeDtypeStruct((n, w), _F32) for w in widths)
    scratch = (
        [pltpu.VMEM((ch,), _I32) for _ in range(4)]
        + [pltpu.VMEM((ch, w), _F32) for w in widths]
        + [pltpu.SemaphoreType.DMA]
    )

    @functools.partial(pl.kernel, mesh=mesh, out_type=out_type,
                       scratch_types=scratch,
                       compiler_params=pltpu.CompilerParams(
                           use_tc_tiling_on_sc=False))
    def body(rid_h, lid_h, eid_h, pid_h, rt_h, lt_h, et_h, pt_h,
             or_h, ol_h, oe_h, op_h, i0, i1, i2, i3, b0, b1, b2, b3, sem):
        wid = lax.axis_index("s") * num_cores + lax.axis_index("c")
        base0 = wid * per_w

        def step(j, carry):
            base = base0 + j * ch
            pltpu.sync_copy(rid_h.at[pl.ds(base, ch)], i0)
            pltpu.sync_copy(lid_h.at[pl.ds(base, ch)], i1)
            pltpu.sync_copy(eid_h.at[pl.ds(base, ch)], i2)
            pltpu.sync_copy(pid_h.at[pl.ds(base, ch)], i3)
            c0 = pltpu.async_copy(rt_h.at[i0], b0, sem)
            c1 = pltpu.async_copy(lt_h.at[i1], b1, sem)
            c2 = pltpu.async_copy(et_h.at[i2], b2, sem)
            c3 = pltpu.async_copy(pt_h.at[i3], b3, sem)
            c0.wait()
            c1.wait()
            c2.wait()
            c3.wait()
            pltpu.sync_copy(b0, or_h.at[pl.ds(base, ch)])
            pltpu.sync_copy(b1, ol_h.at[pl.ds(base, ch)])
            pltpu.sync_copy(b2, oe_h.at[pl.ds(base, ch)])
            pltpu.sync_copy(b3, op_h.at[pl.ds(base, ch)])
            return carry

        lax.fori_loop(0, n_ch, step, 0)

    return body(rid, lid, eid, pid, resp_t, lag_t, el_t, pos_t)


def _epilogue_body(wr_ref, wn_ref, wl_ref, we_ref, wp_ref, nw_ref, nb_ref,
                   lb_ref, g_ref, bb_ref, resp_ref, nm_ref, lag_ref, el_ref,
                   pos_ref, out_ref):
    numemb = (nm_ref[:, 0:1] * nw_ref[0:1, :]
              + nm_ref[:, 1:2] * nw_ref[1:2, :] + nb_ref[...])
    y = jnp.dot(resp_ref[...], wr_ref[...], preferred_element_type=_F32)
    y = y + jnp.dot(numemb, wn_ref[...], preferred_element_type=_F32)
    y = y + jnp.dot(lag_ref[...], wl_ref[...], preferred_element_type=_F32)
    y = y + jnp.dot(el_ref[...], we_ref[...], preferred_element_type=_F32)
    y = y + jnp.dot(pos_ref[...], wp_ref[...], preferred_element_type=_F32)
    y = y + lb_ref[...]
    mu = jnp.mean(y, axis=1, keepdims=True)
    d = y - mu
    var = jnp.mean(d * d, axis=1, keepdims=True)
    out_ref[...] = d / jnp.sqrt(var + 1e-12) * g_ref[...] + bb_ref[...]


def kernel(input_ids, position_ids, timestamp, elapsed_time, response_table,
           num_W, num_b, bn_gamma, bn_beta, bn_mean, bn_var, elapsed_table,
           lag_table, pos_table, lin_W, lin_b, ln_gamma, ln_beta):
    b, l = input_ids.shape
    n = b * l
    hid = lin_W.shape[1]
    resp_w = response_table.shape[1]
    emb = lag_table.shape[1]

    ts_f = timestamp.astype(_F32)
    bn = jnp.stack([bn_mean.astype(_F32), bn_var.astype(_F32),
                    bn_gamma.astype(_F32), bn_beta.astype(_F32)], axis=0)
    rb = 256
    lag_cat, el_cat, n0, n1 = pl.pallas_call(
        _prologue_body,
        grid=(b // rb,),
        in_specs=[
            pl.BlockSpec((4, 2), lambda i: (0, 0)),
            pl.BlockSpec((rb, l), lambda i: (i, 0)),
            pl.BlockSpec((rb, l), lambda i: (i, 0)),
        ],
        out_specs=[pl.BlockSpec((rb, l), lambda i: (i, 0))] * 4,
        out_shape=[
            jax.ShapeDtypeStruct((b, l), _I32),
            jax.ShapeDtypeStruct((b, l), _I32),
            jax.ShapeDtypeStruct((b, l), _F32),
            jax.ShapeDtypeStruct((b, l), _F32),
        ],
    )(bn, ts_f, elapsed_time.astype(_F32))

    rid = input_ids.astype(_I32).reshape(n)
    pid = position_ids.astype(_I32).reshape(n)
    resp_g, lag_g, el_g, pos_g = _sc_gather(
        rid, lag_cat.reshape(n), el_cat.reshape(n), pid,
        response_table, lag_table, elapsed_table, pos_table)

    nm = jnp.stack([n0.reshape(n), n1.reshape(n)], axis=-1)
    wr = lin_W[0:resp_w]
    wn = lin_W[resp_w:resp_w + emb]
    wl = lin_W[resp_w + emb:resp_w + 2 * emb]
    we = lin_W[resp_w + 2 * emb:resp_w + 3 * emb]
    wp = lin_W[resp_w + 3 * emb:resp_w + 4 * emb]

    blk = 512
    const = lambda shape: pl.BlockSpec(shape, lambda i: (0, 0))
    data = lambda w: pl.BlockSpec((blk, w), lambda i: (i, 0))
    weights = (wr, wn, wl, we, wp, num_W, num_b.reshape(1, emb),
               lin_b.reshape(1, hid), ln_gamma.reshape(1, hid),
               ln_beta.reshape(1, hid))
    out = pl.pallas_call(
        _epilogue_body,
        grid=(n // blk,),
        in_specs=[const(w.shape) for w in weights]
        + [data(resp_w), data(2), data(emb), data(emb), data(emb)],
        out_specs=data(hid),
        out_shape=jax.ShapeDtypeStruct((n, hid), _F32),
    )(*weights, resp_g, nm, lag_g, el_g, pos_g)
    return out.reshape(b, l, hid)
